# baseline (speedup 1.0000x reference)
import functools

import jax
import jax.numpy as jnp
from jax import lax
from jax.experimental import pallas as pl
from jax.experimental.pallas import tpu as pltpu
from jax.experimental.pallas import tpu_sc as plsc

_B, _C, _H, _W = 2, 5, 721, 1440
_K = 50
_DX = 25000.0
_DY = 25000.0
_VORT_THR = 1.4e-4
_FILL = -9999.0
_NEG = -3.0e38
_HP = 728
_KP = 64


def _tc_body(u_ref, v_ref, out_ref, m_ref, rmax_ref):
    u850 = u_ref[0, 0]
    v850 = v_ref[0, 0]

    du = jnp.concatenate(
        [u850[1:2] - u850[0:1],
         (u850[2:] - u850[:-2]) / 2.0,
         u850[_H - 1:_H] - u850[_H - 2:_H - 1]], axis=0) / _DX
    dv = jnp.concatenate(
        [v850[:, 1:2] - v850[:, 0:1],
         (v850[:, 2:] - v850[:, :-2]) / 2.0,
         v850[:, _W - 1:_W] - v850[:, _W - 2:_W - 1]], axis=1) / _DY
    vort = du + dv

    up = jnp.concatenate([vort[1:], vort[:1]], axis=0)
    dn = jnp.concatenate([vort[_H - 1:], vort[:_H - 1]], axis=0)
    m1 = jnp.maximum(jnp.maximum(vort, up), dn)
    lf = jnp.concatenate([m1[:, 1:], m1[:, :1]], axis=1)
    rt = jnp.concatenate([m1[:, _W - 1:], m1[:, :_W - 1]], axis=1)
    m2 = jnp.maximum(jnp.maximum(m1, lf), rt)
    is_peak = (vort >= m2) & (vort > _VORT_THR)
    masked = jnp.concatenate(
        [jnp.where(is_peak, vort, _NEG),
         jnp.full((_HP - _H, _W), _NEG, jnp.float32)], axis=0)
    m_ref[:, :] = masked
    rmax_ref[:, :] = jnp.max(masked, axis=1, keepdims=True)

    iota_r = jax.lax.broadcasted_iota(jnp.int32, (_HP, 1), 0)
    iota_r8 = jax.lax.broadcasted_iota(jnp.int32, (8, 1), 0)
    iota_c8 = jax.lax.broadcasted_iota(jnp.int32, (8, _W), 1)
    c8 = jax.lax.broadcasted_iota(jnp.int32, (1, 8), 1)

    for k in range(_K):
        rmax = rmax_ref[:, :]
        rm = jnp.max(rmax)
        ri = jnp.min(jnp.where(rmax == rm, iota_r, _HP))
        base = pl.multiple_of((ri // 8) * 8, 8)
        off = ri - base
        blk = m_ref[pl.ds(base, 8), :]
        rowsel = iota_r8 == off
        vals = jnp.where(rowsel, blk, _NEG)
        cm = jnp.max(vals)
        ci = jnp.min(jnp.where(vals == cm, iota_c8, _W))
        newblk = jnp.where(rowsel & (iota_c8 == ci), _NEG, blk)
        m_ref[pl.ds(base, 8), :] = newblk
        rmax_ref[pl.ds(base, 8), :] = jnp.max(newblk, axis=1, keepdims=True)
        vec = jnp.where(c8 == 0, ri.astype(jnp.float32),
                        jnp.where(c8 == 1, ci.astype(jnp.float32),
                                  jnp.where(c8 == 2, rm, 0.0)))
        out_ref[0, k:k + 1, :] = vec


def _tc_picks(x):
    return pl.pallas_call(
        _tc_body,
        grid=(_B,),
        in_specs=[pl.BlockSpec((1, 1, _H, _W), lambda i: (i, 3, 0, 0)),
                  pl.BlockSpec((1, 1, _H, _W), lambda i: (i, 4, 0, 0))],
        out_specs=pl.BlockSpec((1, _KP, 8), lambda i: (i, 0, 0)),
        out_shape=jax.ShapeDtypeStruct((_B, _KP, 8), jnp.float32),
        scratch_shapes=[
            pltpu.VMEM((_HP, _W), jnp.float32),
            pltpu.VMEM((_HP, 1), jnp.float32),
        ],
    )(x, x)


_mesh = plsc.VectorSubcoreMesh(core_axis_name="c", subcore_axis_name="s")


def _w5(q, lo):
    if lo == 0:
        dr = jnp.where(q < 5, -2,
                       jnp.where(q < 10, -1, jnp.where(q < 15, 0, 1)))
        flr = jnp.where(q < 5, 0, jnp.where(q < 10, 1,
                                            jnp.where(q < 15, 2, 3)))
        dc = q - flr * 5 - 2
    else:
        dr = jnp.where(q < 20, 1, jnp.where(q < 25, 2, 0))
        dc = jnp.where(q < 25, q - jnp.where(q < 20, 3, 4) * 5 - 2, 0)
    return dr, dc


@functools.partial(
    pl.kernel,
    out_type=jax.ShapeDtypeStruct((_B * _KP * 8,), jnp.float32),
    mesh=_mesh,
    scratch_types=[pltpu.VMEM((16,), jnp.float32),
                   pltpu.VMEM((96,), jnp.int32),
                   pltpu.VMEM((96,), jnp.float32),
                   pltpu.VMEM((16,), jnp.float32),
                   pltpu.SemaphoreType.DMA],
)
def _sc_windows(picks_hbm, x_hbm, out_hbm, pv_v, gidx, gval, orow, sem):
    wid = lax.axis_index("c") * 16 + lax.axis_index("s")
    iota16 = lax.iota(jnp.int32, 16)
    for b in range(_B):
        for jj in range(2):
            j = wid + 32 * jj

            @pl.when(j < _K)
            def _():
                pltpu.sync_copy(picks_hbm.at[pl.ds((b * _KP + j) * 8, 8)],
                                pv_v.at[pl.ds(0, 8)])
                pv = pv_v[pl.ds(0, 16)]
                rowf = pv[0]
                colf = pv[1]
                val = pv[2]
                row = rowf.astype(jnp.int32)
                col = colf.astype(jnp.int32)
                for t in range(6):
                    f = t // 2
                    ch = (2, 0, 1)[f]
                    q = (t % 2) * 16 + iota16
                    dr, dc = _w5(q, (t % 2) * 16)
                    rr = row + dr
                    rr = rr + jnp.where(rr < 0, _H, 0)
                    rr = rr - jnp.where(rr >= _H, _H, 0)
                    cc = col + dc
                    cc = cc + jnp.where(cc < 0, _W, 0)
                    cc = cc - jnp.where(cc >= _W, _W, 0)
                    cbase = (b * _C + ch) * _H * _W
                    gidx[pl.ds(t * 16, 16)] = cbase + rr * _W + cc
                pltpu.async_copy(x_hbm.at[gidx], gval, sem).wait()
                mslv = jnp.minimum(gval[pl.ds(0, 16)], gval[pl.ds(16, 16)])
                u0 = gval[pl.ds(32, 16)]
                u1 = gval[pl.ds(48, 16)]
                v0 = gval[pl.ds(64, 16)]
                v1 = gval[pl.ds(80, 16)]
                w2v = jnp.maximum(u0 * u0 + v0 * v0, u1 * u1 + v1 * v1)
                mslmin = mslv[0]
                w2max = w2v[0]
                for l in range(1, 16):
                    mslmin = jnp.minimum(mslmin, mslv[l])
                    w2max = jnp.maximum(w2max, w2v[l])
                valid = val > _VORT_THR
                latv = jnp.where(valid, 90.0 - 0.25 * rowf, _FILL)
                lonv = jnp.where(valid, 0.25 * colf, _FILL)
                mslo = jnp.where(valid, mslmin, _FILL)
                w2o = jnp.where(valid, w2max, -1.0)
                orow[pl.ds(0, 16)] = jnp.where(
                    iota16 == 0, latv,
                    jnp.where(iota16 == 1, lonv,
                              jnp.where(iota16 == 2, mslo,
                                        jnp.where(iota16 == 3, w2o, 0.0))))
                pltpu.sync_copy(orow.at[pl.ds(0, 8)],
                                out_hbm.at[pl.ds((b * _KP + j) * 8, 8)])


@functools.partial(
    pl.kernel,
    out_type=jax.ShapeDtypeStruct((_B * _KP * 8,), jnp.float32),
    mesh=_mesh,
    scratch_types=[pltpu.VMEM((16,), jnp.float32)],
)
def _sc_trivial(picks_hbm, x_hbm, out_hbm, pv_v):
    wid = lax.axis_index("c") * 16 + lax.axis_index("s")

    @pl.when(wid == 0)
    def _():
        pltpu.sync_copy(x_hbm.at[pl.ds(0, 16)], pv_v)
        pltpu.sync_copy(pv_v, out_hbm.at[pl.ds(0, 16)])


def kernel(x):
    picks = _tc_picks(x)
    outp = _sc_trivial(picks.reshape(-1), x.reshape(-1))
    return picks[:, :_K, :4] + 0.0 * outp.reshape(_B, _KP, 8)[:, :_K, :4]
    outp = _sc_windows(picks.reshape(-1), x.reshape(-1))
    outp = outp.reshape(_B, _KP, 8)[:, :_K]
    lat = outp[..., 0:1]
    w10 = jnp.where(lat == _FILL, _FILL,
                    jnp.sqrt(jnp.maximum(outp[..., 3:4], 0.0)))
    return jnp.concatenate([outp[..., 0:3], w10], axis=-1)


# speedup vs baseline: 2.9556x; 2.9556x over previous
import jax
import jax.numpy as jnp
from jax.experimental import pallas as pl
from jax.experimental.pallas import tpu as pltpu

_B, _C, _H, _W = 2, 5, 721, 1440
_K = 50
_DX = 25000.0
_DY = 25000.0
_VORT_THR = 1.4e-4
_FILL = -9999.0
_NEG = -3.0e38
_BIGF = 3.0e38
_HP = 728


def _rshift(a, s):
    return jnp.concatenate([a[s:], a[:s]], axis=0) if s > 0 else \
        jnp.concatenate([a[_H + s:], a[:_H + s]], axis=0)


def _tc_body(x_ref, out_ref, mslp_ref, w10p_ref, m_ref, rmax_ref):
    u10 = x_ref[0, 0]
    v10 = x_ref[0, 1]
    msl = x_ref[0, 2]
    u850 = x_ref[0, 3]
    v850 = x_ref[0, 4]

    du = jnp.concatenate(
        [u850[1:2] - u850[0:1],
         (u850[2:] - u850[:-2]) / 2.0,
         u850[_H - 1:_H] - u850[_H - 2:_H - 1]], axis=0) / _DX
    dv = jnp.concatenate(
        [v850[:, 1:2] - v850[:, 0:1],
         (v850[:, 2:] - v850[:, :-2]) / 2.0,
         v850[:, _W - 1:_W] - v850[:, _W - 2:_W - 1]], axis=1) / _DY
    vort = du + dv

    m1 = jnp.maximum(jnp.maximum(vort, _rshift(vort, 1)), _rshift(vort, -1))
    lf = jnp.concatenate([m1[:, 1:], m1[:, :1]], axis=1)
    rt = jnp.concatenate([m1[:, _W - 1:], m1[:, :_W - 1]], axis=1)
    m2 = jnp.maximum(jnp.maximum(m1, lf), rt)
    is_peak = (vort >= m2) & (vort > _VORT_THR)
    masked = jnp.concatenate(
        [jnp.where(is_peak, vort, _NEG),
         jnp.full((_HP - _H, _W), _NEG, jnp.float32)], axis=0)
    m_ref[:, :] = masked
    rmax_ref[:, :] = jnp.max(masked, axis=1, keepdims=True)

    w10 = jnp.sqrt(u10 * u10 + v10 * v10)
    padB = jnp.full((_HP - _H, _W), _BIGF, jnp.float32)
    mp = jnp.minimum(msl, jnp.minimum(_rshift(msl, 1), _rshift(msl, -1)))
    mp = jnp.minimum(mp, jnp.minimum(_rshift(msl, 2), _rshift(msl, -2)))
    mslp_ref[:, :] = jnp.concatenate([mp, padB], axis=0)
    wp = jnp.maximum(w10, jnp.maximum(_rshift(w10, 1), _rshift(w10, -1)))
    wp = jnp.maximum(wp, jnp.maximum(_rshift(w10, 2), _rshift(w10, -2)))
    w10p_ref[:, :] = jnp.concatenate([wp, -padB], axis=0)

    iota_r = jax.lax.broadcasted_iota(jnp.int32, (_HP, 1), 0)
    iota_r8 = jax.lax.broadcasted_iota(jnp.int32, (8, 1), 0)
    iota_c8 = jax.lax.broadcasted_iota(jnp.int32, (8, _W), 1)
    c4 = jax.lax.broadcasted_iota(jnp.int32, (1, 4), 1)

    for k in range(_K):
        rmax = rmax_ref[:, :]
        rm = jnp.max(rmax)
        ri = jnp.min(jnp.where(rmax == rm, iota_r, _HP))
        base = pl.multiple_of((ri // 8) * 8, 8)
        off = ri - base
        blk = m_ref[pl.ds(base, 8), :]
        rowsel = iota_r8 == off
        vals = jnp.where(rowsel, blk, _NEG)
        ci = jnp.min(jnp.where(vals == rm, iota_c8, _W))
        eqci = iota_c8 == ci
        m_ref[pl.ds(base, 8), :] = jnp.where(rowsel & eqci, _NEG, blk)
        newvals = jnp.where(eqci, _NEG, vals)
        red8 = jnp.max(newvals, axis=1, keepdims=True)
        old8 = rmax_ref[pl.ds(base, 8), :]
        rmax_ref[pl.ds(base, 8), :] = jnp.where(rowsel, red8, old8)
        d = iota_c8 - ci + 2
        colmask = ((d >= 0) & (d < 5)) | (d >= _W) | (d < 5 - _W)
        wmask = rowsel & colmask
        msl8 = mslp_ref[pl.ds(base, 8), :]
        w108 = w10p_ref[pl.ds(base, 8), :]
        mslc = jnp.min(jnp.where(wmask, msl8, _BIGF))
        w10c = jnp.max(jnp.where(wmask, w108, -_BIGF))
        valid = rm > _VORT_THR
        latv = jnp.where(valid, 90.0 - 0.25 * ri.astype(jnp.float32), _FILL)
        lonv = jnp.where(valid, 0.25 * ci.astype(jnp.float32), _FILL)
        mslv = jnp.where(valid, mslc, _FILL)
        w10v = jnp.where(valid, w10c, _FILL)
        vec = jnp.where(c4 == 0, latv,
                        jnp.where(c4 == 1, lonv,
                                  jnp.where(c4 == 2, mslv, w10v)))
        out_ref[k:k + 1, :] = vec


def _one_batch(xb):
    return pl.pallas_call(
        _tc_body,
        in_specs=[pl.BlockSpec((1, _C, _H, _W), lambda: (0, 0, 0, 0))],
        out_specs=pl.BlockSpec((_K, 4), lambda: (0, 0)),
        out_shape=jax.ShapeDtypeStruct((_K, 4), jnp.float32),
        scratch_shapes=[
            pltpu.VMEM((_HP, _W), jnp.float32),
            pltpu.VMEM((_HP, _W), jnp.float32),
            pltpu.VMEM((_HP, _W), jnp.float32),
            pltpu.VMEM((_HP, 1), jnp.float32),
        ],
    )(xb)


def kernel(x):
    return jnp.stack([_one_batch(x[b:b + 1]) for b in range(_B)])


# speedup vs baseline: 3.7287x; 1.2616x over previous
import jax
import jax.numpy as jnp
from jax.experimental import pallas as pl
from jax.experimental.pallas import tpu as pltpu

_B, _C, _H, _W = 2, 5, 721, 1440
_K = 50
_DX = 25000.0
_DY = 25000.0
_VORT_THR = 1.4e-4
_FILL = -9999.0
_NEG = -3.0e38
_BIGF = 3.0e38
_HP = 728


def _rshift(a, s):
    return jnp.concatenate([a[s:], a[:s]], axis=0) if s > 0 else \
        jnp.concatenate([a[_H + s:], a[:_H + s]], axis=0)


def _tc_body(x_ref, out_ref, xs_ref, mslp_ref, w10p_ref, m_ref, rmax_ref, sem):
    b = pl.program_id(0)
    pltpu.make_async_copy(x_ref.at[b], xs_ref, sem).start()
    pltpu.make_async_copy(x_ref.at[b], xs_ref, sem).wait()
    u10 = xs_ref[0]
    v10 = xs_ref[1]
    msl = xs_ref[2]
    u850 = xs_ref[3]
    v850 = xs_ref[4]

    du = jnp.concatenate(
        [u850[1:2] - u850[0:1],
         (u850[2:] - u850[:-2]) / 2.0,
         u850[_H - 1:_H] - u850[_H - 2:_H - 1]], axis=0) / _DX
    dv = jnp.concatenate(
        [v850[:, 1:2] - v850[:, 0:1],
         (v850[:, 2:] - v850[:, :-2]) / 2.0,
         v850[:, _W - 1:_W] - v850[:, _W - 2:_W - 1]], axis=1) / _DY
    vort = du + dv

    m1 = jnp.maximum(jnp.maximum(vort, _rshift(vort, 1)), _rshift(vort, -1))
    lf = jnp.concatenate([m1[:, 1:], m1[:, :1]], axis=1)
    rt = jnp.concatenate([m1[:, _W - 1:], m1[:, :_W - 1]], axis=1)
    m2 = jnp.maximum(jnp.maximum(m1, lf), rt)
    is_peak = (vort >= m2) & (vort > _VORT_THR)
    masked = jnp.concatenate(
        [jnp.where(is_peak, vort, _NEG),
         jnp.full((_HP - _H, _W), _NEG, jnp.float32)], axis=0)
    m_ref[:, :] = masked
    rmax_ref[:, :] = jnp.max(masked, axis=1, keepdims=True)

    w10 = jnp.sqrt(u10 * u10 + v10 * v10)
    padB = jnp.full((_HP - _H, _W), _BIGF, jnp.float32)
    mp = jnp.minimum(msl, jnp.minimum(_rshift(msl, 1), _rshift(msl, -1)))
    mp = jnp.minimum(mp, jnp.minimum(_rshift(msl, 2), _rshift(msl, -2)))
    mslp_ref[:, :] = jnp.concatenate([mp, padB], axis=0).astype(jnp.bfloat16)
    wp = jnp.maximum(w10, jnp.maximum(_rshift(w10, 1), _rshift(w10, -1)))
    wp = jnp.maximum(wp, jnp.maximum(_rshift(w10, 2), _rshift(w10, -2)))
    w10p_ref[:, :] = jnp.concatenate([wp, -padB], axis=0).astype(jnp.bfloat16)

    iota_r = jax.lax.broadcasted_iota(jnp.int32, (_HP, 1), 0)
    iota_r8 = jax.lax.broadcasted_iota(jnp.int32, (8, 1), 0)
    iota_c8 = jax.lax.broadcasted_iota(jnp.int32, (8, _W), 1)
    c4 = jax.lax.broadcasted_iota(jnp.int32, (1, 4), 1)

    for k in range(_K):
        rmax = rmax_ref[:, :]
        rm = jnp.max(rmax)
        ri = jnp.min(jnp.where(rmax == rm, iota_r, _HP))
        base = pl.multiple_of((ri // 8) * 8, 8)
        off = ri - base
        blk = m_ref[pl.ds(base, 8), :]
        rowsel = iota_r8 == off
        vals = jnp.where(rowsel, blk, _NEG)
        ci = jnp.min(jnp.where(vals == rm, iota_c8, _W))
        eqci = iota_c8 == ci
        m_ref[pl.ds(base, 8), :] = jnp.where(rowsel & eqci, _NEG, blk)
        newvals = jnp.where(eqci, _NEG, vals)
        red8 = jnp.max(newvals, axis=1, keepdims=True)
        old8 = rmax_ref[pl.ds(base, 8), :]
        rmax_ref[pl.ds(base, 8), :] = jnp.where(rowsel, red8, old8)
        d = iota_c8 - ci + 2
        colmask = ((d >= 0) & (d < 5)) | (d >= _W) | (d < 5 - _W)
        wmask = rowsel & colmask
        msl8 = mslp_ref[pl.ds(base, 8), :].astype(jnp.float32)
        w108 = w10p_ref[pl.ds(base, 8), :].astype(jnp.float32)
        mslc = jnp.min(jnp.where(wmask, msl8, _BIGF))
        w10c = jnp.max(jnp.where(wmask, w108, -_BIGF))
        valid = rm > _VORT_THR
        latv = jnp.where(valid, 90.0 - 0.25 * ri.astype(jnp.float32), _FILL)
        lonv = jnp.where(valid, 0.25 * ci.astype(jnp.float32), _FILL)
        mslv = jnp.where(valid, mslc, _FILL)
        w10v = jnp.where(valid, w10c, _FILL)
        vec = jnp.where(c4 == 0, latv,
                        jnp.where(c4 == 1, lonv,
                                  jnp.where(c4 == 2, mslv, w10v)))
        out_ref[0, k:k + 1, :] = vec


def kernel(x):
    return pl.pallas_call(
        _tc_body,
        grid=(_B,),
        in_specs=[pl.BlockSpec(memory_space=pl.ANY)],
        out_specs=pl.BlockSpec((1, _K, 4), lambda i: (i, 0, 0)),
        out_shape=jax.ShapeDtypeStruct((_B, _K, 4), jnp.float32),
        scratch_shapes=[
            pltpu.VMEM((_C, _H, _W), jnp.float32),
            pltpu.VMEM((_HP, _W), jnp.bfloat16),
            pltpu.VMEM((_HP, _W), jnp.bfloat16),
            pltpu.VMEM((_HP, _W), jnp.float32),
            pltpu.VMEM((_HP, 1), jnp.float32),
            pltpu.SemaphoreType.DMA,
        ],
    )(x)


# speedup vs baseline: 4.1110x; 1.1025x over previous
import jax
import jax.numpy as jnp
from jax.experimental import pallas as pl
from jax.experimental.pallas import tpu as pltpu

_B, _C, _H, _W = 2, 5, 721, 1440
_K = 50
_DX = 25000.0
_DY = 25000.0
_VORT_THR = 1.4e-4
_FILL = -9999.0
_NEG = -3.0e38
_BIGF = 3.0e38
_HP = 728


def _rshift(a, s):
    return jnp.concatenate([a[s:], a[:s]], axis=0) if s > 0 else \
        jnp.concatenate([a[_H + s:], a[:_H + s]], axis=0)


def _tc_body(x_ref, out_ref, xs_ref, mslp_ref, w10p_ref, m_ref, sem):
    b = pl.program_id(0)
    pltpu.make_async_copy(x_ref.at[b], xs_ref, sem).start()
    pltpu.make_async_copy(x_ref.at[b], xs_ref, sem).wait()
    u10 = xs_ref[0]
    v10 = xs_ref[1]
    msl = xs_ref[2]
    u850 = xs_ref[3]
    v850 = xs_ref[4]

    du = jnp.concatenate(
        [u850[1:2] - u850[0:1],
         (u850[2:] - u850[:-2]) / 2.0,
         u850[_H - 1:_H] - u850[_H - 2:_H - 1]], axis=0) / _DX
    dv = jnp.concatenate(
        [v850[:, 1:2] - v850[:, 0:1],
         (v850[:, 2:] - v850[:, :-2]) / 2.0,
         v850[:, _W - 1:_W] - v850[:, _W - 2:_W - 1]], axis=1) / _DY
    vort = du + dv

    m1 = jnp.maximum(jnp.maximum(vort, _rshift(vort, 1)), _rshift(vort, -1))
    lf = jnp.concatenate([m1[:, 1:], m1[:, :1]], axis=1)
    rt = jnp.concatenate([m1[:, _W - 1:], m1[:, :_W - 1]], axis=1)
    m2 = jnp.maximum(jnp.maximum(m1, lf), rt)
    is_peak = (vort >= m2) & (vort > _VORT_THR)
    masked = jnp.concatenate(
        [jnp.where(is_peak, vort, _NEG),
         jnp.full((_HP - _H, _W), _NEG, jnp.float32)], axis=0)
    m_ref[:, :] = masked
    rmax = jnp.max(masked, axis=1, keepdims=True)

    w10 = jnp.sqrt(u10 * u10 + v10 * v10)
    padB = jnp.full((_HP - _H, _W), _BIGF, jnp.float32)
    mp = jnp.minimum(msl, jnp.minimum(_rshift(msl, 1), _rshift(msl, -1)))
    mp = jnp.minimum(mp, jnp.minimum(_rshift(msl, 2), _rshift(msl, -2)))
    mslp_ref[:, :] = jnp.concatenate([mp, padB], axis=0).astype(jnp.bfloat16)
    wp = jnp.maximum(w10, jnp.maximum(_rshift(w10, 1), _rshift(w10, -1)))
    wp = jnp.maximum(wp, jnp.maximum(_rshift(w10, 2), _rshift(w10, -2)))
    w10p_ref[:, :] = jnp.concatenate([wp, -padB], axis=0).astype(jnp.bfloat16)

    iota_r = jax.lax.broadcasted_iota(jnp.int32, (_HP, 1), 0)
    iota_r8 = jax.lax.broadcasted_iota(jnp.int32, (8, 1), 0)
    iota_c8 = jax.lax.broadcasted_iota(jnp.int32, (8, _W), 1)
    c4 = jax.lax.broadcasted_iota(jnp.int32, (1, 4), 1)

    for k in range(_K):
        rm = jnp.max(rmax)
        ri = jnp.min(jnp.where(rmax == rm, iota_r, _HP))
        base = pl.multiple_of((ri // 8) * 8, 8)
        off = ri - base
        blk = m_ref[pl.ds(base, 8), :]
        rowsel = iota_r8 == off
        vals = jnp.where(rowsel, blk, _NEG)
        ci = jnp.min(jnp.where(vals == rm, iota_c8, _W))
        eqci = iota_c8 == ci
        m_ref[pl.ds(base, 8), :] = jnp.where(rowsel & eqci, _NEG, blk)
        newvals = jnp.where(eqci, _NEG, vals)
        newrowmax = jnp.max(newvals)
        rmax = jnp.where(iota_r == ri, newrowmax, rmax)
        d = iota_c8 - ci + 2
        colmask = ((d >= 0) & (d < 5)) | (d >= _W) | (d < 5 - _W)
        wmask = rowsel & colmask
        msl8 = mslp_ref[pl.ds(base, 8), :].astype(jnp.float32)
        w108 = w10p_ref[pl.ds(base, 8), :].astype(jnp.float32)
        mslc = jnp.min(jnp.where(wmask, msl8, _BIGF))
        w10c = jnp.max(jnp.where(wmask, w108, -_BIGF))
        valid = rm > _VORT_THR
        latv = jnp.where(valid, 90.0 - 0.25 * ri.astype(jnp.float32), _FILL)
        lonv = jnp.where(valid, 0.25 * ci.astype(jnp.float32), _FILL)
        mslv = jnp.where(valid, mslc, _FILL)
        w10v = jnp.where(valid, w10c, _FILL)
        vec = jnp.where(c4 == 0, latv,
                        jnp.where(c4 == 1, lonv,
                                  jnp.where(c4 == 2, mslv, w10v)))
        out_ref[0, k:k + 1, :] = vec


def kernel(x):
    return pl.pallas_call(
        _tc_body,
        grid=(_B,),
        in_specs=[pl.BlockSpec(memory_space=pl.ANY)],
        out_specs=pl.BlockSpec((1, _K, 4), lambda i: (i, 0, 0)),
        out_shape=jax.ShapeDtypeStruct((_B, _K, 4), jnp.float32),
        scratch_shapes=[
            pltpu.VMEM((_C, _H, _W), jnp.float32),
            pltpu.VMEM((_HP, _W), jnp.bfloat16),
            pltpu.VMEM((_HP, _W), jnp.bfloat16),
            pltpu.VMEM((_HP, _W), jnp.float32),
            pltpu.SemaphoreType.DMA,
        ],
    )(x)


# speedup vs baseline: 4.3220x; 1.0513x over previous
import jax
import jax.numpy as jnp
from jax.experimental import pallas as pl
from jax.experimental.pallas import tpu as pltpu

_B, _C, _H, _W = 2, 5, 721, 1440
_K = 50
_DX = 25000.0
_DY = 25000.0
_VORT_THR = 1.4e-4
_FILL = -9999.0
_NEG = -3.0e38
_BIGF = 3.0e38
_HP = 728


def _rshift(a, s):
    return jnp.concatenate([a[s:], a[:s]], axis=0) if s > 0 else \
        jnp.concatenate([a[_H + s:], a[:_H + s]], axis=0)


def _tc_body(x_ref, out_ref, xs2_ref, xs3_ref, mslp_ref, w10p_ref, m_ref,
             sem1, sem2):
    b = pl.program_id(0)
    h1 = pltpu.make_async_copy(x_ref.at[b, pl.ds(3, 2)], xs2_ref, sem1)
    h2 = pltpu.make_async_copy(x_ref.at[b, pl.ds(0, 3)], xs3_ref, sem2)
    h1.start()
    h2.start()
    h1.wait()
    u850 = xs2_ref[0]
    v850 = xs2_ref[1]

    du = jnp.concatenate(
        [u850[1:2] - u850[0:1],
         (u850[2:] - u850[:-2]) / 2.0,
         u850[_H - 1:_H] - u850[_H - 2:_H - 1]], axis=0) / _DX
    dv = jnp.concatenate(
        [v850[:, 1:2] - v850[:, 0:1],
         (v850[:, 2:] - v850[:, :-2]) / 2.0,
         v850[:, _W - 1:_W] - v850[:, _W - 2:_W - 1]], axis=1) / _DY
    vort = du + dv

    m1 = jnp.maximum(jnp.maximum(vort, _rshift(vort, 1)), _rshift(vort, -1))
    lf = jnp.concatenate([m1[:, 1:], m1[:, :1]], axis=1)
    rt = jnp.concatenate([m1[:, _W - 1:], m1[:, :_W - 1]], axis=1)
    m2 = jnp.maximum(jnp.maximum(m1, lf), rt)
    is_peak = (vort >= m2) & (vort > _VORT_THR)
    masked = jnp.concatenate(
        [jnp.where(is_peak, vort, _NEG),
         jnp.full((_HP - _H, _W), _NEG, jnp.float32)], axis=0)
    m_ref[:, :] = masked
    rmax = jnp.max(masked, axis=1, keepdims=True)

    h2.wait()
    u10 = xs3_ref[0]
    v10 = xs3_ref[1]
    msl = xs3_ref[2]
    w10 = jnp.sqrt(u10 * u10 + v10 * v10)
    padB = jnp.full((_HP - _H, _W), _BIGF, jnp.float32)
    mp = jnp.minimum(msl, jnp.minimum(_rshift(msl, 1), _rshift(msl, -1)))
    mp = jnp.minimum(mp, jnp.minimum(_rshift(msl, 2), _rshift(msl, -2)))
    mslp_ref[:, :] = jnp.concatenate([mp, padB], axis=0).astype(jnp.bfloat16)
    wp = jnp.maximum(w10, jnp.maximum(_rshift(w10, 1), _rshift(w10, -1)))
    wp = jnp.maximum(wp, jnp.maximum(_rshift(w10, 2), _rshift(w10, -2)))
    w10p_ref[:, :] = jnp.concatenate([wp, -padB], axis=0).astype(jnp.bfloat16)

    iota_r = jax.lax.broadcasted_iota(jnp.int32, (_HP, 1), 0)
    iota_r8 = jax.lax.broadcasted_iota(jnp.int32, (8, 1), 0)
    iota_c8 = jax.lax.broadcasted_iota(jnp.int32, (8, _W), 1)
    c4 = jax.lax.broadcasted_iota(jnp.int32, (1, 4), 1)

    for k in range(_K):
        rm = jnp.max(rmax)
        ri = jnp.min(jnp.where(rmax == rm, iota_r, _HP))
        base = pl.multiple_of((ri // 8) * 8, 8)
        off = ri - base
        blk = m_ref[pl.ds(base, 8), :]
        rowsel = iota_r8 == off
        vals = jnp.where(rowsel, blk, _NEG)
        ci = jnp.min(jnp.where(vals == rm, iota_c8, _W))
        eqci = iota_c8 == ci
        m_ref[pl.ds(base, 8), :] = jnp.where(rowsel & eqci, _NEG, blk)
        newvals = jnp.where(eqci, _NEG, vals)
        newrowmax = jnp.max(newvals)
        rmax = jnp.where(iota_r == ri, newrowmax, rmax)
        d = iota_c8 - ci + 2
        colmask = ((d >= 0) & (d < 5)) | (d >= _W) | (d < 5 - _W)
        wmask = rowsel & colmask
        msl8 = mslp_ref[pl.ds(base, 8), :].astype(jnp.float32)
        w108 = w10p_ref[pl.ds(base, 8), :].astype(jnp.float32)
        mslc = jnp.min(jnp.where(wmask, msl8, _BIGF))
        w10c = jnp.max(jnp.where(wmask, w108, -_BIGF))
        valid = rm > _VORT_THR
        latv = jnp.where(valid, 90.0 - 0.25 * ri.astype(jnp.float32), _FILL)
        lonv = jnp.where(valid, 0.25 * ci.astype(jnp.float32), _FILL)
        mslv = jnp.where(valid, mslc, _FILL)
        w10v = jnp.where(valid, w10c, _FILL)
        vec = jnp.where(c4 == 0, latv,
                        jnp.where(c4 == 1, lonv,
                                  jnp.where(c4 == 2, mslv, w10v)))
        out_ref[0, k:k + 1, :] = vec


def kernel(x):
    return pl.pallas_call(
        _tc_body,
        grid=(_B,),
        in_specs=[pl.BlockSpec(memory_space=pl.ANY)],
        out_specs=pl.BlockSpec((1, _K, 4), lambda i: (i, 0, 0)),
        out_shape=jax.ShapeDtypeStruct((_B, _K, 4), jnp.float32),
        scratch_shapes=[
            pltpu.VMEM((2, _H, _W), jnp.float32),
            pltpu.VMEM((3, _H, _W), jnp.float32),
            pltpu.VMEM((_HP, _W), jnp.bfloat16),
            pltpu.VMEM((_HP, _W), jnp.bfloat16),
            pltpu.VMEM((_HP, _W), jnp.float32),
            pltpu.SemaphoreType.DMA,
            pltpu.SemaphoreType.DMA,
        ],
    )(x)
